# baseline (speedup 1.0000x reference)
import functools

import jax
import jax.numpy as jnp
from jax import lax
from jax.experimental import pallas as pl
from jax.experimental.pallas import tpu as pltpu
from jax.experimental.pallas import tpu_sc as plsc

_SEP = 16
_EPS = 1e-12
_NW = 32


def _gather_rows(table, ids):
    nb = ids.shape[0]
    d = table.shape[1]
    b_per_w = nb // _NW
    mesh = plsc.VectorSubcoreMesh(core_axis_name="c", subcore_axis_name="s")

    @functools.partial(
        pl.kernel,
        mesh=mesh,
        out_type=jax.ShapeDtypeStruct((nb, d), jnp.float32),
        scratch_types=[
            pltpu.VMEM((b_per_w,), jnp.int32),
            pltpu.VMEM((b_per_w, d), jnp.float32),
            pltpu.SemaphoreType.DMA,
        ],
    )
    def k(table_hbm, idx_hbm, out_hbm, idx_v, rows_v, sem):
        wid = lax.axis_index("s") * 2 + lax.axis_index("c")
        base = wid * b_per_w
        pltpu.sync_copy(idx_hbm.at[pl.ds(base, b_per_w)], idx_v)
        pltpu.async_copy(table_hbm.at[idx_v], rows_v, sem).wait()
        pltpu.sync_copy(rows_v, out_hbm.at[pl.ds(base, b_per_w)])

    return k(table, ids)


def _dense_body(gx_ref, mask_ref, ln1w_ref, ln1b_ref, fcw_ref, fcb_ref,
                ln2w_ref, ln2b_ref, pe_ref, out_ref):
    l, nseg = gx_ref.shape[1], pe_ref.shape[0]
    wvec = ln1w_ref.shape[1]
    x = gx_ref[0][:, :wvec]
    u = jnp.mean(x, axis=1, keepdims=True)
    xc = x - u
    v = jnp.mean(xc * xc, axis=1, keepdims=True)
    h = xc * lax.rsqrt(v + _EPS) * ln1w_ref[...] + ln1b_ref[...]
    y = lax.dot_general(h, fcw_ref[...], (((1,), (0,)), ((), ())),
                        preferred_element_type=jnp.float32,
                        precision=lax.Precision.HIGHEST)
    y = jnp.maximum(y + fcb_ref[...], 0.0)
    u2 = jnp.mean(y, axis=1, keepdims=True)
    yc = y - u2
    v2 = jnp.mean(yc * yc, axis=1, keepdims=True)
    z = yc * lax.rsqrt(v2 + _EPS) * ln2w_ref[...] + ln2b_ref[...]
    row = lax.broadcasted_iota(jnp.int32, (nseg, l), 0)
    col = lax.broadcasted_iota(jnp.int32, (nseg, l), 1)
    keep = ((col // _SEP == row) & (col % _SEP != _SEP - 1)
            & (mask_ref[0] != 1))
    pool = jnp.where(keep, 1.0 / (_SEP - 1), 0.0)
    seg = lax.dot_general(pool, z, (((1,), (0,)), ((), ())),
                          preferred_element_type=jnp.float32,
                          precision=lax.Precision.HIGHEST)
    out_ref[0] = seg + pe_ref[...]


def _dense(g3, mask3, ln1w, ln1b, fcwt, fcb, ln2w, ln2b, pe_seg):
    b, l, dpad = g3.shape
    wvec = fcwt.shape[0]
    hid = fcwt.shape[1]
    nseg = pe_seg.shape[0]
    return pl.pallas_call(
        _dense_body,
        grid=(b,),
        in_specs=[
            pl.BlockSpec((1, l, dpad), lambda i: (i, 0, 0)),
            pl.BlockSpec((1, 1, l), lambda i: (i, 0, 0)),
            pl.BlockSpec((1, wvec), lambda i: (0, 0)),
            pl.BlockSpec((1, wvec), lambda i: (0, 0)),
            pl.BlockSpec((wvec, hid), lambda i: (0, 0)),
            pl.BlockSpec((1, hid), lambda i: (0, 0)),
            pl.BlockSpec((1, hid), lambda i: (0, 0)),
            pl.BlockSpec((1, hid), lambda i: (0, 0)),
            pl.BlockSpec((nseg, hid), lambda i: (0, 0)),
        ],
        out_specs=pl.BlockSpec((1, nseg, hid), lambda i: (i, 0, 0)),
        out_shape=jax.ShapeDtypeStruct((b, nseg, hid), jnp.float32),
    )(g3, mask3, ln1w, ln1b, fcwt, fcb, ln2w, ln2b, pe_seg)


def kernel(ingr_input_ids, ingr_sep_masks, emb_table, ln1_w, ln1_b,
           fc_W, fc_b, ln2_w, ln2_b, pe):
    b, l = ingr_input_ids.shape
    wvec = emb_table.shape[1]
    hid = fc_W.shape[0]
    nseg = l // _SEP
    dpad = -(-wvec // 128) * 128
    ids = ingr_input_ids.reshape(-1).astype(jnp.int32)
    table_pad = jnp.pad(emb_table.astype(jnp.float32), ((0, 0), (0, dpad - wvec)))
    gathered = _gather_rows(table_pad, ids)
    out = _dense(
        gathered.reshape(b, l, dpad),
        ingr_sep_masks.astype(jnp.int32).reshape(b, 1, l),
        ln1_w.reshape(1, wvec), ln1_b.reshape(1, wvec),
        fc_W.T, fc_b.reshape(1, hid),
        ln2_w.reshape(1, hid), ln2_b.reshape(1, hid),
        pe[:nseg],
    )
    return out


# speedup vs baseline: 1.8804x; 1.8804x over previous
import functools

import jax
import jax.numpy as jnp
from jax import lax
from jax.experimental import pallas as pl
from jax.experimental.pallas import tpu as pltpu
from jax.experimental.pallas import tpu_sc as plsc

_SEP = 16
_EPS = 1e-12
_NW = 32


def _gather_rows(table, ids):
    nb = ids.shape[0]
    d = table.shape[1]
    b_per_w = nb // _NW
    mesh = plsc.VectorSubcoreMesh(core_axis_name="c", subcore_axis_name="s")

    @functools.partial(
        pl.kernel,
        mesh=mesh,
        out_type=jax.ShapeDtypeStruct((nb, d), jnp.float32),
        scratch_types=[
            pltpu.VMEM((b_per_w,), jnp.int32),
            pltpu.VMEM((b_per_w, d), jnp.float32),
            pltpu.SemaphoreType.DMA,
        ],
    )
    def k(table_hbm, idx_hbm, out_hbm, idx_v, rows_v, sem):
        wid = lax.axis_index("s") * 2 + lax.axis_index("c")
        base = wid * b_per_w
        pltpu.sync_copy(idx_hbm.at[pl.ds(base, b_per_w)], idx_v)
        pltpu.async_copy(table_hbm.at[idx_v], rows_v, sem).wait()
        pltpu.sync_copy(rows_v, out_hbm.at[pl.ds(base, b_per_w)])

    return k(table, ids)


def _pad_body(src_ref, dst_ref):
    w = src_ref.shape[1]
    dst_ref[:, :w] = src_ref[...]
    dst_ref[:, w:] = jnp.zeros_like(dst_ref[:, w:])


def _pad_table(table, dpad):
    v, w = table.shape
    blk = 2048
    grid = -(-v // blk)
    return pl.pallas_call(
        _pad_body,
        grid=(grid,),
        in_specs=[pl.BlockSpec((blk, w), lambda i: (i, 0))],
        out_specs=pl.BlockSpec((blk, dpad), lambda i: (i, 0)),
        out_shape=jax.ShapeDtypeStruct((v, dpad), jnp.float32),
    )(table)


def _dense_body(gx_ref, mask_ref, ln1w_ref, ln1b_ref, fcw_ref, fcb_ref,
                ln2w_ref, ln2b_ref, pe_ref, out_ref):
    l, nseg = gx_ref.shape[1], pe_ref.shape[0]
    wvec = ln1w_ref.shape[1]
    x = gx_ref[0][:, :wvec]
    u = jnp.mean(x, axis=1, keepdims=True)
    xc = x - u
    v = jnp.mean(xc * xc, axis=1, keepdims=True)
    h = xc * lax.rsqrt(v + _EPS) * ln1w_ref[...] + ln1b_ref[...]
    y = lax.dot_general(h, fcw_ref[...], (((1,), (0,)), ((), ())),
                        preferred_element_type=jnp.float32,
                        precision=lax.Precision.HIGHEST)
    y = jnp.maximum(y + fcb_ref[...], 0.0)
    u2 = jnp.mean(y, axis=1, keepdims=True)
    yc = y - u2
    v2 = jnp.mean(yc * yc, axis=1, keepdims=True)
    z = yc * lax.rsqrt(v2 + _EPS) * ln2w_ref[...] + ln2b_ref[...]
    row = lax.broadcasted_iota(jnp.int32, (nseg, l), 0)
    col = lax.broadcasted_iota(jnp.int32, (nseg, l), 1)
    keep = ((col // _SEP == row) & (col % _SEP != _SEP - 1)
            & (mask_ref[0] != 1))
    pool = jnp.where(keep, 1.0 / (_SEP - 1), 0.0)
    seg = lax.dot_general(pool, z, (((1,), (0,)), ((), ())),
                          preferred_element_type=jnp.float32,
                          precision=lax.Precision.HIGHEST)
    out_ref[0] = seg + pe_ref[...]


def _dense(g3, mask3, ln1w, ln1b, fcwt, fcb, ln2w, ln2b, pe_seg):
    b, l, dpad = g3.shape
    wvec = fcwt.shape[0]
    hid = fcwt.shape[1]
    nseg = pe_seg.shape[0]
    return pl.pallas_call(
        _dense_body,
        grid=(b,),
        in_specs=[
            pl.BlockSpec((1, l, dpad), lambda i: (i, 0, 0)),
            pl.BlockSpec((1, 1, l), lambda i: (i, 0, 0)),
            pl.BlockSpec((1, wvec), lambda i: (0, 0)),
            pl.BlockSpec((1, wvec), lambda i: (0, 0)),
            pl.BlockSpec((wvec, hid), lambda i: (0, 0)),
            pl.BlockSpec((1, hid), lambda i: (0, 0)),
            pl.BlockSpec((1, hid), lambda i: (0, 0)),
            pl.BlockSpec((1, hid), lambda i: (0, 0)),
            pl.BlockSpec((nseg, hid), lambda i: (0, 0)),
        ],
        out_specs=pl.BlockSpec((1, nseg, hid), lambda i: (i, 0, 0)),
        out_shape=jax.ShapeDtypeStruct((b, nseg, hid), jnp.float32),
    )(g3, mask3, ln1w, ln1b, fcwt, fcb, ln2w, ln2b, pe_seg)


def kernel(ingr_input_ids, ingr_sep_masks, emb_table, ln1_w, ln1_b,
           fc_W, fc_b, ln2_w, ln2_b, pe):
    b, l = ingr_input_ids.shape
    wvec = emb_table.shape[1]
    hid = fc_W.shape[0]
    nseg = l // _SEP
    dpad = -(-wvec // 128) * 128
    ids = ingr_input_ids.reshape(-1).astype(jnp.int32)
    table_pad = _pad_table(emb_table.astype(jnp.float32), dpad)
    gathered = _gather_rows(table_pad, ids)
    out = _dense(
        gathered.reshape(b, l, dpad),
        ingr_sep_masks.astype(jnp.int32).reshape(b, 1, l),
        ln1_w.reshape(1, wvec), ln1_b.reshape(1, wvec),
        fc_W.T, fc_b.reshape(1, hid),
        ln2_w.reshape(1, hid), ln2_b.reshape(1, hid),
        pe[:nseg],
    )
    return out


# speedup vs baseline: 2.2888x; 1.2172x over previous
import functools

import jax
import jax.numpy as jnp
from jax import lax
from jax.experimental import pallas as pl
from jax.experimental.pallas import tpu as pltpu
from jax.experimental.pallas import tpu_sc as plsc

_SEP = 16
_EPS = 1e-12
_NW = 32


def _gather_rows(table, ids):
    nb = ids.shape[0]
    d = table.shape[1]
    b_per_w = nb // _NW
    mesh = plsc.VectorSubcoreMesh(core_axis_name="c", subcore_axis_name="s")

    @functools.partial(
        pl.kernel,
        mesh=mesh,
        out_type=jax.ShapeDtypeStruct((nb, d), jnp.float32),
        scratch_types=[
            pltpu.VMEM((b_per_w,), jnp.int32),
            pltpu.VMEM((b_per_w, d), jnp.float32),
            pltpu.SemaphoreType.DMA,
        ],
        compiler_params=pltpu.CompilerParams(needs_layout_passes=False),
    )
    def k(table_hbm, idx_hbm, out_hbm, idx_v, rows_v, sem):
        wid = lax.axis_index("s") * 2 + lax.axis_index("c")
        base = wid * b_per_w
        pltpu.sync_copy(idx_hbm.at[pl.ds(base, b_per_w)], idx_v)
        lane = lax.iota(jnp.int32, 16)
        for c in range(b_per_w // 16):
            vals = idx_v[pl.ds(c * 16, 16)]
            copies = []
            for j in range(16):
                s = jnp.sum(jnp.where(lane == j, vals, 0))
                copies.append(pltpu.async_copy(
                    table_hbm.at[pl.ds(s, 1), :],
                    rows_v.at[pl.ds(c * 16 + j, 1), :], sem))
            for cp in copies:
                cp.wait()
        pltpu.sync_copy(rows_v, out_hbm.at[pl.ds(base, b_per_w)])

    return k(table, ids)


def _dense_body(gx_ref, mask_ref, ln1w_ref, ln1b_ref, fcw_ref, fcb_ref,
                ln2w_ref, ln2b_ref, pe_ref, out_ref):
    l, nseg = gx_ref.shape[1], pe_ref.shape[0]
    wvec = ln1w_ref.shape[1]
    x = gx_ref[0][:, :wvec]
    u = jnp.mean(x, axis=1, keepdims=True)
    xc = x - u
    v = jnp.mean(xc * xc, axis=1, keepdims=True)
    h = xc * lax.rsqrt(v + _EPS) * ln1w_ref[...] + ln1b_ref[...]
    y = lax.dot_general(h, fcw_ref[...], (((1,), (0,)), ((), ())),
                        preferred_element_type=jnp.float32,
                        precision=lax.Precision.HIGHEST)
    y = jnp.maximum(y + fcb_ref[...], 0.0)
    u2 = jnp.mean(y, axis=1, keepdims=True)
    yc = y - u2
    v2 = jnp.mean(yc * yc, axis=1, keepdims=True)
    z = yc * lax.rsqrt(v2 + _EPS) * ln2w_ref[...] + ln2b_ref[...]
    row = lax.broadcasted_iota(jnp.int32, (nseg, l), 0)
    col = lax.broadcasted_iota(jnp.int32, (nseg, l), 1)
    keep = ((col // _SEP == row) & (col % _SEP != _SEP - 1)
            & (mask_ref[0] != 1))
    pool = jnp.where(keep, 1.0 / (_SEP - 1), 0.0)
    seg = lax.dot_general(pool, z, (((1,), (0,)), ((), ())),
                          preferred_element_type=jnp.float32,
                          precision=lax.Precision.HIGHEST)
    out_ref[0] = seg + pe_ref[...]


def _dense(g3, mask3, ln1w, ln1b, fcwt, fcb, ln2w, ln2b, pe_seg):
    b, l, dpad = g3.shape
    wvec = fcwt.shape[0]
    hid = fcwt.shape[1]
    nseg = pe_seg.shape[0]
    return pl.pallas_call(
        _dense_body,
        grid=(b,),
        in_specs=[
            pl.BlockSpec((1, l, dpad), lambda i: (i, 0, 0)),
            pl.BlockSpec((1, 1, l), lambda i: (i, 0, 0)),
            pl.BlockSpec((1, wvec), lambda i: (0, 0)),
            pl.BlockSpec((1, wvec), lambda i: (0, 0)),
            pl.BlockSpec((wvec, hid), lambda i: (0, 0)),
            pl.BlockSpec((1, hid), lambda i: (0, 0)),
            pl.BlockSpec((1, hid), lambda i: (0, 0)),
            pl.BlockSpec((1, hid), lambda i: (0, 0)),
            pl.BlockSpec((nseg, hid), lambda i: (0, 0)),
        ],
        out_specs=pl.BlockSpec((1, nseg, hid), lambda i: (i, 0, 0)),
        out_shape=jax.ShapeDtypeStruct((b, nseg, hid), jnp.float32),
    )(g3, mask3, ln1w, ln1b, fcwt, fcb, ln2w, ln2b, pe_seg)


def kernel(ingr_input_ids, ingr_sep_masks, emb_table, ln1_w, ln1_b,
           fc_W, fc_b, ln2_w, ln2_b, pe):
    b, l = ingr_input_ids.shape
    wvec = emb_table.shape[1]
    hid = fc_W.shape[0]
    nseg = l // _SEP
    ids = ingr_input_ids.reshape(-1).astype(jnp.int32)
    gathered = _gather_rows(emb_table.astype(jnp.float32), ids)
    out = _dense(
        gathered.reshape(b, l, wvec),
        ingr_sep_masks.astype(jnp.int32).reshape(b, 1, l),
        ln1_w.reshape(1, wvec), ln1_b.reshape(1, wvec),
        fc_W.T, fc_b.reshape(1, hid),
        ln2_w.reshape(1, hid), ln2_b.reshape(1, hid),
        pe[:nseg],
    )
    return out


# speedup vs baseline: 3.1244x; 1.3651x over previous
import functools

import jax
import jax.numpy as jnp
from jax import lax
from jax.experimental import pallas as pl
from jax.experimental.pallas import tpu as pltpu
from jax.experimental.pallas import tpu_sc as plsc

_SEP = 16
_EPS = 1e-12
_NW = 32


def _gather_rows(table, ids):
    nb = ids.shape[0] * ids.shape[1]
    d = table.shape[1]
    b_per_w = nb // _NW
    mesh = plsc.VectorSubcoreMesh(core_axis_name="c", subcore_axis_name="s")

    @functools.partial(
        pl.kernel,
        mesh=mesh,
        out_type=jax.ShapeDtypeStruct((nb, d), jnp.float32),
        scratch_types=[
            pltpu.VMEM((b_per_w,), jnp.int32),
            pltpu.VMEM((b_per_w, d), jnp.float32),
            pltpu.SemaphoreType.DMA,
        ],
        compiler_params=pltpu.CompilerParams(needs_layout_passes=False),
    )
    def k(table_hbm, idx_hbm, out_hbm, idx_v, rows_v, sem):
        wid = lax.axis_index("s") * 2 + lax.axis_index("c")
        base = wid * b_per_w
        l = idx_hbm.shape[1]
        pltpu.sync_copy(
            idx_hbm.at[base // l, pl.ds(base % l, b_per_w)], idx_v)
        lane = lax.iota(jnp.int32, 16)
        for c in range(b_per_w // 16):
            vals = idx_v[pl.ds(c * 16, 16)]
            copies = []
            for j in range(16):
                s = jnp.sum(jnp.where(lane == j, vals, 0))
                copies.append(pltpu.async_copy(
                    table_hbm.at[pl.ds(s, 1), :],
                    rows_v.at[pl.ds(c * 16 + j, 1), :], sem))
            for cp in copies:
                cp.wait()
        pltpu.sync_copy(rows_v, out_hbm.at[pl.ds(base, b_per_w)])

    return k(table, ids)


def _dense_body(gx_ref, mask_ref, ln1w_ref, ln1b_ref, fcw_ref, fcb_ref,
                ln2w_ref, ln2b_ref, pe_ref, out_ref):
    l, nseg = gx_ref.shape[1], pe_ref.shape[0]
    wvec = ln1w_ref.shape[1]
    x = gx_ref[0][:, :wvec]
    u = jnp.mean(x, axis=1, keepdims=True)
    xc = x - u
    v = jnp.mean(xc * xc, axis=1, keepdims=True)
    h = xc * lax.rsqrt(v + _EPS) * ln1w_ref[...] + ln1b_ref[...]
    y = lax.dot_general(h, fcw_ref[...], (((1,), (1,)), ((), ())),
                        preferred_element_type=jnp.float32,
                        precision=lax.Precision.DEFAULT)
    y = jnp.maximum(y + fcb_ref[...], 0.0)
    u2 = jnp.mean(y, axis=1, keepdims=True)
    yc = y - u2
    v2 = jnp.mean(yc * yc, axis=1, keepdims=True)
    z = yc * lax.rsqrt(v2 + _EPS) * ln2w_ref[...] + ln2b_ref[...]
    row = lax.broadcasted_iota(jnp.int32, (nseg, l), 0)
    col = lax.broadcasted_iota(jnp.int32, (nseg, l), 1)
    keep = ((col // _SEP == row) & (col % _SEP != _SEP - 1)
            & (mask_ref[0] != 1))
    pool = jnp.where(keep, 1.0 / (_SEP - 1), 0.0)
    seg = lax.dot_general(pool, z, (((1,), (0,)), ((), ())),
                          preferred_element_type=jnp.float32,
                          precision=lax.Precision.DEFAULT)
    out_ref[0] = seg + pe_ref[...]


def _dense(g3, mask3, ln1w, ln1b, fcw, fcb, ln2w, ln2b, pe, nseg):
    b, l, dpad = g3.shape
    wvec = fcw.shape[1]
    hid = fcw.shape[0]
    return pl.pallas_call(
        _dense_body,
        grid=(b,),
        in_specs=[
            pl.BlockSpec((1, l, dpad), lambda i: (i, 0, 0)),
            pl.BlockSpec((1, 1, l), lambda i: (i, 0, 0)),
            pl.BlockSpec((1, wvec), lambda i: (0, 0)),
            pl.BlockSpec((1, wvec), lambda i: (0, 0)),
            pl.BlockSpec((hid, wvec), lambda i: (0, 0)),
            pl.BlockSpec((1, hid), lambda i: (0, 0)),
            pl.BlockSpec((1, hid), lambda i: (0, 0)),
            pl.BlockSpec((1, hid), lambda i: (0, 0)),
            pl.BlockSpec((nseg, hid), lambda i: (0, 0)),
        ],
        out_specs=pl.BlockSpec((1, nseg, hid), lambda i: (i, 0, 0)),
        out_shape=jax.ShapeDtypeStruct((b, nseg, hid), jnp.float32),
    )(g3, mask3, ln1w, ln1b, fcw, fcb, ln2w, ln2b, pe)


def kernel(ingr_input_ids, ingr_sep_masks, emb_table, ln1_w, ln1_b,
           fc_W, fc_b, ln2_w, ln2_b, pe):
    b, l = ingr_input_ids.shape
    wvec = emb_table.shape[1]
    hid = fc_W.shape[0]
    nseg = l // _SEP
    ids = ingr_input_ids.astype(jnp.int32)
    gathered = _gather_rows(emb_table.astype(jnp.float32), ids)
    out = _dense(
        gathered.reshape(b, l, wvec),
        ingr_sep_masks.astype(jnp.int32).reshape(b, 1, l),
        ln1_w.reshape(1, wvec), ln1_b.reshape(1, wvec),
        fc_W, fc_b.reshape(1, hid),
        ln2_w.reshape(1, hid), ln2_b.reshape(1, hid),
        pe, nseg,
    )
    return out


# speedup vs baseline: 3.2494x; 1.0400x over previous
import functools

import jax
import jax.numpy as jnp
from jax import lax
from jax.experimental import pallas as pl
from jax.experimental.pallas import tpu as pltpu
from jax.experimental.pallas import tpu_sc as plsc

_SEP = 16
_EPS = 1e-12
_NW = 32


def _gather_rows(table, ids):
    nb = ids.shape[0] * ids.shape[1]
    d = table.shape[1]
    b_per_w = nb // _NW
    mesh = plsc.VectorSubcoreMesh(core_axis_name="c", subcore_axis_name="s")

    @functools.partial(
        pl.kernel,
        mesh=mesh,
        out_type=jax.ShapeDtypeStruct((nb, d), jnp.float32),
        scratch_types=[
            pltpu.VMEM((b_per_w,), jnp.int32),
            pltpu.VMEM((b_per_w, d), jnp.float32),
            pltpu.SemaphoreType.DMA,
        ],
        compiler_params=pltpu.CompilerParams(needs_layout_passes=False),
    )
    def k(table_hbm, idx_hbm, out_hbm, idx_v, rows_v, sem):
        wid = lax.axis_index("s") * 2 + lax.axis_index("c")
        base = wid * b_per_w
        l = idx_hbm.shape[1]
        pltpu.sync_copy(
            idx_hbm.at[base // l, pl.ds(base % l, b_per_w)], idx_v)
        lane = lax.iota(jnp.int32, 16)

        def fire(c):
            vals = idx_v[pl.ds(c * 16, 16)]
            return [pltpu.async_copy(
                table_hbm.at[pl.ds(jnp.sum(jnp.where(lane == j, vals, 0)), 1), :],
                rows_v.at[pl.ds(c * 16 + j, 1), :], sem)
                for j in range(16)]

        nchunk = b_per_w // 16
        prev = fire(0)
        for c in range(1, nchunk):
            cur = fire(c)
            for cp in prev:
                cp.wait()
            prev = cur
        for cp in prev:
            cp.wait()
        pltpu.sync_copy(rows_v, out_hbm.at[pl.ds(base, b_per_w)])

    return k(table, ids)


def _dense_body(gx_ref, mask_ref, ln1w_ref, ln1b_ref, fcw_ref, fcb_ref,
                ln2w_ref, ln2b_ref, pe_ref, out_ref):
    l, nseg = gx_ref.shape[1], pe_ref.shape[0]
    wvec = ln1w_ref.shape[1]
    x = gx_ref[0][:, :wvec]
    u = jnp.mean(x, axis=1, keepdims=True)
    xc = x - u
    v = jnp.mean(xc * xc, axis=1, keepdims=True)
    h = xc * lax.rsqrt(v + _EPS) * ln1w_ref[...] + ln1b_ref[...]
    y = lax.dot_general(h, fcw_ref[...], (((1,), (1,)), ((), ())),
                        preferred_element_type=jnp.float32,
                        precision=lax.Precision.DEFAULT)
    y = jnp.maximum(y + fcb_ref[...], 0.0)
    u2 = jnp.mean(y, axis=1, keepdims=True)
    yc = y - u2
    v2 = jnp.mean(yc * yc, axis=1, keepdims=True)
    z = yc * lax.rsqrt(v2 + _EPS) * ln2w_ref[...] + ln2b_ref[...]
    row = lax.broadcasted_iota(jnp.int32, (nseg, l), 0)
    col = lax.broadcasted_iota(jnp.int32, (nseg, l), 1)
    keep = ((col // _SEP == row) & (col % _SEP != _SEP - 1)
            & (mask_ref[pl.ds(pl.program_id(0), 1), :] != 1))
    pool = jnp.where(keep, 1.0 / (_SEP - 1), 0.0)
    seg = lax.dot_general(pool, z, (((1,), (0,)), ((), ())),
                          preferred_element_type=jnp.float32,
                          precision=lax.Precision.DEFAULT)
    out_ref[0] = seg + pe_ref[...]


def _dense(g3, mask2, ln1w, ln1b, fcw, fcb, ln2w, ln2b, pe, nseg):
    b, l, dpad = g3.shape
    wvec = fcw.shape[1]
    hid = fcw.shape[0]
    return pl.pallas_call(
        _dense_body,
        grid=(b,),
        in_specs=[
            pl.BlockSpec((1, l, dpad), lambda i: (i, 0, 0)),
            pl.BlockSpec((b, l), lambda i: (0, 0)),
            pl.BlockSpec((1, wvec), lambda i: (0, 0)),
            pl.BlockSpec((1, wvec), lambda i: (0, 0)),
            pl.BlockSpec((hid, wvec), lambda i: (0, 0)),
            pl.BlockSpec((1, hid), lambda i: (0, 0)),
            pl.BlockSpec((1, hid), lambda i: (0, 0)),
            pl.BlockSpec((1, hid), lambda i: (0, 0)),
            pl.BlockSpec((nseg, hid), lambda i: (0, 0)),
        ],
        out_specs=pl.BlockSpec((1, nseg, hid), lambda i: (i, 0, 0)),
        out_shape=jax.ShapeDtypeStruct((b, nseg, hid), jnp.float32),
    )(g3, mask2, ln1w, ln1b, fcw, fcb, ln2w, ln2b, pe)


def kernel(ingr_input_ids, ingr_sep_masks, emb_table, ln1_w, ln1_b,
           fc_W, fc_b, ln2_w, ln2_b, pe):
    b, l = ingr_input_ids.shape
    wvec = emb_table.shape[1]
    hid = fc_W.shape[0]
    nseg = l // _SEP
    ids = ingr_input_ids.astype(jnp.int32)
    gathered = _gather_rows(emb_table.astype(jnp.float32), ids)
    out = _dense(
        gathered.reshape(b, l, wvec),
        ingr_sep_masks.astype(jnp.int32),
        ln1_w.reshape(1, wvec), ln1_b.reshape(1, wvec),
        fc_W, fc_b.reshape(1, hid),
        ln2_w.reshape(1, hid), ln2_b.reshape(1, hid),
        pe, nseg,
    )
    return out
